# baseline (speedup 1.0000x reference)
import jax
import jax.numpy as jnp
from jax.experimental import pallas as pl
from jax.experimental.pallas import tpu as pltpu

_B, _N, _E, _H = 128, 2048, 128, 128
_STEPS = 16
_NT = 128
_NTILES = _N // _NT
_NRES = 5
_NS = _NTILES - _NRES
_f32 = jnp.float32
_bf16 = jnp.bfloat16


def _is_res(n):
    return jnp.logical_and(n % 3 == 1, n < 15)


def _stream_ord(n):
    return n - (n + 2) // 3


def _node_tile(n):
    return jnp.where(_is_res(n), _NS + n // 3, _stream_ord(n))


def _kproj_body(wk_ref, enc_ref, os_ref, or_ref):
    i = pl.program_id(0)
    m = enc_ref[...].reshape(_B * _NT, _E).astype(_bf16)
    o = jax.lax.dot_general(m, wk_ref[...], (((1,), (0,)), ((), ())),
                            preferred_element_type=_f32)
    o = o.reshape(_B, _NT, _H)

    @pl.when(i < _NS)
    def _stream_part():
        os_ref[...] = o

    @pl.when(i >= _NS)
    def _resident_part():
        or_ref[...] = o[None]


def _each_row_copy(enc_ref, start_scr, idx_smem, gsem, do_wait):
    def body(b, carry):
        ib = idx_smem[b, 0]
        cp = pltpu.make_async_copy(enc_ref.at[b, ib], start_scr.at[b], gsem)
        if do_wait:
            cp.wait()
        else:
            cp.start()
        return carry
    jax.lax.fori_loop(0, _B, body, 0)


def _dec_body(ks_ref, kr_ref, g_ref, vb_ref, wih_ref, whh_ref, wq_ref,
              bih_ref, bhh_ref, enc_ref, act_ref, logp_ref,
              h_scr, c_scr, start_scr, q_scr, s_scr, vis_scr, hh_scr,
              msg_scr, arg_scr, ms_scr, se_scr, sel_scr, lacc_scr, pidx_scr,
              idx_smem, gsem):
    t = pl.program_id(0)
    n = pl.program_id(1)

    @pl.when(n == 0)
    def _step_start():
        @pl.when(t == 0)
        def _init():
            c_scr[...] = jnp.zeros((_B, _H), _f32)
            start_scr[...] = jnp.zeros((_B, _E), _f32)
            hh_scr[...] = jnp.zeros((_B, 4 * _H), _f32)
            lacc_scr[...] = jnp.zeros((_B, 1), _f32)

        @pl.when(t > 0)
        def _wait_gathers():
            _each_row_copy(enc_ref, start_scr, idx_smem, gsem, do_wait=True)

        x = start_scr[...]
        c = c_scr[...]
        gates = (jax.lax.dot_general(x.astype(_bf16), wih_ref[...],
                                     (((1,), (0,)), ((), ())),
                                     preferred_element_type=_f32)
                 + bih_ref[...]
                 + hh_scr[...]
                 + bhh_ref[...])
        gi, gf, gg, go = jnp.split(gates, 4, axis=-1)
        gi = jax.nn.sigmoid(gi)
        gf = jax.nn.sigmoid(gf)
        gg = jnp.tanh(gg)
        go = jax.nn.sigmoid(go)
        c2 = gf * c + gi * gg
        h2 = go * jnp.tanh(c2)
        c_scr[...] = c2
        q_scr[...] = jax.lax.dot_general(h2.astype(_bf16), wq_ref[...],
                                         (((1,), (0,)), ((), ())),
                                         preferred_element_type=_f32)
        hh_scr[...] = jax.lax.dot_general(h2.astype(_bf16), whh_ref[...],
                                          (((1,), (0,)), ((), ())),
                                          preferred_element_type=_f32)
        msg_scr[...] = jnp.full((_B, 1), -3.4e38, _f32)
        ms_scr[...] = jnp.full((_B, 1), -3.4e38, _f32)
        se_scr[...] = jnp.zeros((_B, 1), _f32)
        sel_scr[...] = jnp.zeros((_B, 1), _f32)
        arg_scr[...] = jnp.zeros((_B, 1), jnp.int32)

    def _tile_scores(kt):
        th = jnp.tanh(q_scr[...][:, None, :] + kt)
        thb = th.astype(_bf16).reshape(_B * _NT, _H)
        s1 = jax.lax.dot_general(vb_ref[...], thb, (((1,), (1,)), ((), ())),
                                 preferred_element_type=_f32)
        return s1.reshape(_B, _NT)

    nt_id = _node_tile(n)

    @pl.when(jnp.logical_not(_is_res(n)))
    def _streamed_tile():
        s_scr[...] = _tile_scores(ks_ref[...])

    @pl.when(_is_res(n))
    def _resident_tile():
        s_scr[...] = _tile_scores(kr_ref[n // 3])

    s = s_scr[...]
    lane = jax.lax.broadcasted_iota(jnp.int32, (_B, _NT), 1) + nt_id * _NT
    vt = vis_scr[nt_id]
    vt = jnp.where(lane == pidx_scr[...], 1.0, vt)
    vt = jnp.where(t == 0, 0.0, vt)
    vis_scr[nt_id] = vt
    s = s - 1000000.0 * vt
    sg = g_ref[0] + s

    mt = jnp.max(sg, axis=-1, keepdims=True)
    la = jnp.min(jnp.where(sg == mt, lane, _N), axis=-1, keepdims=True)
    mst = jnp.max(s, axis=-1, keepdims=True)
    set_ = jnp.sum(jnp.exp(s - mst), axis=-1, keepdims=True)
    selt = jnp.sum(jnp.where(lane == la, s, 0.0), axis=-1, keepdims=True)

    m0 = msg_scr[...]
    better = mt > m0
    arg_scr[...] = jnp.where(better, la, arg_scr[...])
    sel_scr[...] = jnp.where(better, selt, sel_scr[...])
    msg_scr[...] = jnp.maximum(m0, mt)
    ms0 = ms_scr[...]
    msn = jnp.maximum(ms0, mst)
    se_scr[...] = (se_scr[...] * jnp.exp(ms0 - msn)
                   + set_ * jnp.exp(mst - msn))
    ms_scr[...] = msn

    @pl.when(n == _NTILES - 1)
    def _step_end():
        idx = arg_scr[...]
        act_ref[...] = idx[None, :, :]
        p = jnp.exp(sel_scr[...] - ms_scr[...]) / se_scr[...]
        lacc = lacc_scr[...] + p
        lacc_scr[...] = lacc
        pidx_scr[...] = idx

        @pl.when(t == _STEPS - 1)
        def _finish():
            logp_ref[...] = lacc

        @pl.when(t < _STEPS - 1)
        def _gather_next():
            pltpu.sync_copy(arg_scr, idx_smem)
            _each_row_copy(enc_ref, start_scr, idx_smem, gsem, do_wait=False)


def kernel(encoder_output, W_ih, W_hh, b_ih, b_hh, W_q, W_k, v, max_steps):
    del max_steps
    base = jax.random.key(42)
    g_all = jnp.stack(
        [jax.random.gumbel(jax.random.fold_in(base, t), (_B, _N), _f32)
         for t in range(_STEPS)])
    wih_t = W_ih.T.astype(_bf16)
    whh_t = W_hh.T.astype(_bf16)
    wq_t = W_q.T.astype(_bf16)
    vb = v.astype(_bf16)[None, :]
    bih2 = b_ih[None, :]
    bhh2 = b_hh[None, :]

    k_stream, k_res = pl.pallas_call(
        _kproj_body,
        grid=(_NTILES,),
        in_specs=[
            pl.BlockSpec((_E, _H), lambda i: (0, 0)),
            pl.BlockSpec((_B, _NT, _E), lambda i: (0, i, 0)),
        ],
        out_specs=[
            pl.BlockSpec((_B, _NT, _H),
                         lambda i: (0, jnp.minimum(i, _NS - 1), 0)),
            pl.BlockSpec((1, _B, _NT, _H),
                         lambda i: (jnp.maximum(i - _NS, 0), 0, 0, 0)),
        ],
        out_shape=[
            jax.ShapeDtypeStruct((_B, _NS * _NT, _H), _f32),
            jax.ShapeDtypeStruct((_NRES, _B, _NT, _H), _f32),
        ],
        compiler_params=pltpu.CompilerParams(
            vmem_limit_bytes=56 * 1024 * 1024),
    )(W_k.T.astype(_bf16), encoder_output)

    act, logp = pl.pallas_call(
        _dec_body,
        grid=(_STEPS, _NTILES),
        in_specs=[
            pl.BlockSpec((_B, _NT, _H),
                         lambda t, n: (0, _stream_ord(n), 0)),
            pl.BlockSpec((_NRES, _B, _NT, _H), lambda t, n: (0, 0, 0, 0)),
            pl.BlockSpec((1, _B, _NT), lambda t, n: (t, 0, _node_tile(n))),
            pl.BlockSpec((1, _H), lambda t, n: (0, 0)),
            pl.BlockSpec((_E, 4 * _H), lambda t, n: (0, 0)),
            pl.BlockSpec((_H, 4 * _H), lambda t, n: (0, 0)),
            pl.BlockSpec((_H, _H), lambda t, n: (0, 0)),
            pl.BlockSpec((1, 4 * _H), lambda t, n: (0, 0)),
            pl.BlockSpec((1, 4 * _H), lambda t, n: (0, 0)),
            pl.BlockSpec(memory_space=pltpu.MemorySpace.HBM),
        ],
        out_specs=[
            pl.BlockSpec((1, _B, 1), lambda t, n: (t, 0, 0)),
            pl.BlockSpec((_B, 1), lambda t, n: (0, 0)),
        ],
        out_shape=[
            jax.ShapeDtypeStruct((_STEPS, _B, 1), jnp.int32),
            jax.ShapeDtypeStruct((_B, 1), _f32),
        ],
        scratch_shapes=[
            pltpu.VMEM((_B, _H), _f32),
            pltpu.VMEM((_B, _H), _f32),
            pltpu.VMEM((_B, _E), _f32),
            pltpu.VMEM((_B, _H), _f32),
            pltpu.VMEM((_B, _NT), _f32),
            pltpu.VMEM((_NTILES, _B, _NT), _f32),
            pltpu.VMEM((_B, 4 * _H), _f32),
            pltpu.VMEM((_B, 1), _f32),
            pltpu.VMEM((_B, 1), jnp.int32),
            pltpu.VMEM((_B, 1), _f32),
            pltpu.VMEM((_B, 1), _f32),
            pltpu.VMEM((_B, 1), _f32),
            pltpu.VMEM((_B, 1), _f32),
            pltpu.VMEM((_B, 1), jnp.int32),
            pltpu.SMEM((_B, 1), jnp.int32),
            pltpu.SemaphoreType.DMA,
        ],
        compiler_params=pltpu.CompilerParams(
            vmem_limit_bytes=61 * 1024 * 1024),
    )(k_stream, k_res, g_all, vb, wih_t, whh_t, wq_t,
      bih2, bhh2, encoder_output)

    return (act[:, :, 0].T, logp[:, 0])


# speedup vs baseline: 1.0720x; 1.0720x over previous
import jax
import jax.numpy as jnp
from jax.experimental import pallas as pl
from jax.experimental.pallas import tpu as pltpu

_B, _N, _E, _H = 128, 2048, 128, 128
_STEPS = 16
_NT = 128
_NTILES = _N // _NT
_NRES = 5
_NS = _NTILES - _NRES
_f32 = jnp.float32
_bf16 = jnp.bfloat16


def _kproj_body(wk_ref, enc_ref, os_ref, or_ref):
    i = pl.program_id(0)
    m = enc_ref[...].reshape(_B * _NT, _E).astype(_bf16)
    o = jax.lax.dot_general(m, wk_ref[...], (((1,), (0,)), ((), ())),
                            preferred_element_type=_f32)
    o = o.reshape(_B, _NT, _H)

    @pl.when(i < _NS)
    def _stream_part():
        os_ref[...] = o

    @pl.when(i >= _NS)
    def _resident_part():
        or_ref[...] = o[None]


def _each_row_copy(enc_ref, start_scr, idx_smem, gsem, do_wait):
    def body(b, carry):
        ib = idx_smem[b, 0]
        cp = pltpu.make_async_copy(enc_ref.at[b, ib], start_scr.at[b], gsem)
        if do_wait:
            cp.wait()
        else:
            cp.start()
        return carry
    jax.lax.fori_loop(0, _B, body, 0)


def _dec_body(ks_ref, kr_ref, g_ref, vb_ref, wih_ref, whh_ref, wq_ref,
              bih_ref, bhh_ref, enc_ref, act_ref, logp_ref,
              h_scr, c_scr, start_scr, q_scr, s_scr, vis_scr, hh_scr,
              msg_scr, arg_scr, ms_scr, se_scr, sel_scr, lacc_scr, pidx_scr,
              idx_smem, gsem):
    t = pl.program_id(0)
    n = pl.program_id(1)

    @pl.when(n == 0)
    def _step_start():
        @pl.when(t == 0)
        def _init():
            c_scr[...] = jnp.zeros((_B, _H), _f32)
            start_scr[...] = jnp.zeros((_B, _E), _f32)
            hh_scr[...] = jnp.zeros((_B, 4 * _H), _f32)
            lacc_scr[...] = jnp.zeros((_B, 1), _f32)

        @pl.when(t > 0)
        def _wait_gathers():
            _each_row_copy(enc_ref, start_scr, idx_smem, gsem, do_wait=True)

        x = start_scr[...]
        c = c_scr[...]
        gates = (jax.lax.dot_general(x.astype(_bf16), wih_ref[...],
                                     (((1,), (0,)), ((), ())),
                                     preferred_element_type=_f32)
                 + bih_ref[...]
                 + hh_scr[...]
                 + bhh_ref[...])
        gi, gf, gg, go = jnp.split(gates, 4, axis=-1)
        gi = jax.nn.sigmoid(gi)
        gf = jax.nn.sigmoid(gf)
        gg = jnp.tanh(gg)
        go = jax.nn.sigmoid(go)
        c2 = gf * c + gi * gg
        h2 = go * jnp.tanh(c2)
        c_scr[...] = c2
        q_scr[...] = jax.lax.dot_general(h2.astype(_bf16), wq_ref[...],
                                         (((1,), (0,)), ((), ())),
                                         preferred_element_type=_f32)
        hh_scr[...] = jax.lax.dot_general(h2.astype(_bf16), whh_ref[...],
                                          (((1,), (0,)), ((), ())),
                                          preferred_element_type=_f32)
        msg_scr[...] = jnp.full((_B, 1), -3.4e38, _f32)
        ms_scr[...] = jnp.full((_B, 1), -3.4e38, _f32)
        se_scr[...] = jnp.zeros((_B, 1), _f32)
        sel_scr[...] = jnp.zeros((_B, 1), _f32)
        arg_scr[...] = jnp.zeros((_B, 1), jnp.int32)

    def _tile_scores(kt):
        th = jnp.tanh(q_scr[...][:, None, :] + kt)
        thb = th.astype(_bf16).reshape(_B * _NT, _H)
        s1 = jax.lax.dot_general(vb_ref[...], thb, (((1,), (1,)), ((), ())),
                                 preferred_element_type=_f32)
        return s1.reshape(_B, _NT)

    @pl.when(n < _NS)
    def _streamed_tile():
        s_scr[...] = _tile_scores(ks_ref[...])

    @pl.when(n >= _NS)
    def _resident_tile():
        s_scr[...] = _tile_scores(kr_ref[n - _NS])

    s = s_scr[...]
    lane = jax.lax.broadcasted_iota(jnp.int32, (_B, _NT), 1) + n * _NT
    vt = vis_scr[n]
    vt = jnp.where(lane == pidx_scr[...], 1.0, vt)
    vt = jnp.where(t == 0, 0.0, vt)
    vis_scr[n] = vt
    s = s - 1000000.0 * vt
    sg = g_ref[0] + s

    mt = jnp.max(sg, axis=-1, keepdims=True)
    la = jnp.min(jnp.where(sg == mt, lane, _N), axis=-1, keepdims=True)
    mst = jnp.max(s, axis=-1, keepdims=True)
    set_ = jnp.sum(jnp.exp(s - mst), axis=-1, keepdims=True)
    selt = jnp.sum(jnp.where(lane == la, s, 0.0), axis=-1, keepdims=True)

    m0 = msg_scr[...]
    better = mt > m0
    arg_scr[...] = jnp.where(better, la, arg_scr[...])
    sel_scr[...] = jnp.where(better, selt, sel_scr[...])
    msg_scr[...] = jnp.maximum(m0, mt)
    ms0 = ms_scr[...]
    msn = jnp.maximum(ms0, mst)
    se_scr[...] = (se_scr[...] * jnp.exp(ms0 - msn)
                   + set_ * jnp.exp(mst - msn))
    ms_scr[...] = msn

    @pl.when(n == _NTILES - 1)
    def _step_end():
        idx = arg_scr[...]
        act_ref[...] = idx[None, :, :]
        p = jnp.exp(sel_scr[...] - ms_scr[...]) / se_scr[...]
        lacc = lacc_scr[...] + p
        lacc_scr[...] = lacc
        pidx_scr[...] = idx

        @pl.when(t == _STEPS - 1)
        def _finish():
            logp_ref[...] = lacc

        @pl.when(t < _STEPS - 1)
        def _gather_next():
            pltpu.sync_copy(arg_scr, idx_smem)
            _each_row_copy(enc_ref, start_scr, idx_smem, gsem, do_wait=False)


def kernel(encoder_output, W_ih, W_hh, b_ih, b_hh, W_q, W_k, v, max_steps):
    del max_steps
    base = jax.random.key(42)
    g_all = jnp.stack(
        [jax.random.gumbel(jax.random.fold_in(base, t), (_B, _N), _f32)
         for t in range(_STEPS)])
    wih_t = W_ih.T.astype(_bf16)
    whh_t = W_hh.T.astype(_bf16)
    wq_t = W_q.T.astype(_bf16)
    vb = v.astype(_bf16)[None, :]
    bih2 = b_ih[None, :]
    bhh2 = b_hh[None, :]

    k_stream, k_res = pl.pallas_call(
        _kproj_body,
        grid=(_NTILES,),
        in_specs=[
            pl.BlockSpec((_E, _H), lambda i: (0, 0)),
            pl.BlockSpec((_B, _NT, _E), lambda i: (0, i, 0)),
        ],
        out_specs=[
            pl.BlockSpec((_B, _NT, _H),
                         lambda i: (0, jnp.minimum(i, _NS - 1), 0)),
            pl.BlockSpec((1, _B, _NT, _H),
                         lambda i: (jnp.maximum(i - _NS, 0), 0, 0, 0)),
        ],
        out_shape=[
            jax.ShapeDtypeStruct((_B, _NS * _NT, _H), _f32),
            jax.ShapeDtypeStruct((_NRES, _B, _NT, _H), _f32),
        ],
        compiler_params=pltpu.CompilerParams(
            vmem_limit_bytes=56 * 1024 * 1024),
    )(W_k.T.astype(_bf16), encoder_output)

    act, logp = pl.pallas_call(
        _dec_body,
        grid=(_STEPS, _NTILES),
        in_specs=[
            pl.BlockSpec((_B, _NT, _H),
                         lambda t, n: (0, jnp.minimum(n, _NS - 1), 0)),
            pl.BlockSpec((_NRES, _B, _NT, _H), lambda t, n: (0, 0, 0, 0)),
            pl.BlockSpec((1, _B, _NT), lambda t, n: (t, 0, n)),
            pl.BlockSpec((1, _H), lambda t, n: (0, 0)),
            pl.BlockSpec((_E, 4 * _H), lambda t, n: (0, 0)),
            pl.BlockSpec((_H, 4 * _H), lambda t, n: (0, 0)),
            pl.BlockSpec((_H, _H), lambda t, n: (0, 0)),
            pl.BlockSpec((1, 4 * _H), lambda t, n: (0, 0)),
            pl.BlockSpec((1, 4 * _H), lambda t, n: (0, 0)),
            pl.BlockSpec(memory_space=pltpu.MemorySpace.HBM),
        ],
        out_specs=[
            pl.BlockSpec((1, _B, 1), lambda t, n: (t, 0, 0)),
            pl.BlockSpec((_B, 1), lambda t, n: (0, 0)),
        ],
        out_shape=[
            jax.ShapeDtypeStruct((_STEPS, _B, 1), jnp.int32),
            jax.ShapeDtypeStruct((_B, 1), _f32),
        ],
        scratch_shapes=[
            pltpu.VMEM((_B, _H), _f32),
            pltpu.VMEM((_B, _H), _f32),
            pltpu.VMEM((_B, _E), _f32),
            pltpu.VMEM((_B, _H), _f32),
            pltpu.VMEM((_B, _NT), _f32),
            pltpu.VMEM((_NTILES, _B, _NT), _f32),
            pltpu.VMEM((_B, 4 * _H), _f32),
            pltpu.VMEM((_B, 1), _f32),
            pltpu.VMEM((_B, 1), jnp.int32),
            pltpu.VMEM((_B, 1), _f32),
            pltpu.VMEM((_B, 1), _f32),
            pltpu.VMEM((_B, 1), _f32),
            pltpu.VMEM((_B, 1), _f32),
            pltpu.VMEM((_B, 1), jnp.int32),
            pltpu.SMEM((_B, 1), jnp.int32),
            pltpu.SemaphoreType.DMA,
        ],
        compiler_params=pltpu.CompilerParams(
            vmem_limit_bytes=61 * 1024 * 1024),
    )(k_stream, k_res, g_all, vb, wih_t, whh_t, wq_t,
      bih2, bhh2, encoder_output)

    return (act[:, :, 0].T, logp[:, 0])
